# baseline (speedup 1.0000x reference)
import functools

import jax
import jax.numpy as jnp
from jax import lax
from jax.experimental import pallas as pl
from jax.experimental.pallas import tpu as pltpu
from jax.experimental.pallas import tpu_sc as plsc

SEQ_LEN, BATCH, DIM = 200, 4096, 64
TOTAL = SEQ_LEN * BATCH
NC, NS = 2, 16
NW = NC * NS
CHUNK = 256
CPR = BATCH // CHUNK
NCHUNK = TOTAL // CHUNK
C_PER_W = NCHUNK // NW
NPAIR = C_PER_W // 2

_mesh = plsc.VectorSubcoreMesh(core_axis_name="c", subcore_axis_name="s")


@functools.partial(
    pl.kernel,
    out_type=jax.ShapeDtypeStruct((SEQ_LEN, DIM, BATCH), jnp.float32),
    mesh=_mesh,
    scratch_types=[
        pltpu.VMEM((CHUNK,), jnp.int32),
        pltpu.VMEM((CHUNK,), jnp.int32),
        pltpu.VMEM((CHUNK, DIM), jnp.float32),
        pltpu.VMEM((CHUNK, DIM), jnp.float32),
        pltpu.VMEM((DIM, CHUNK), jnp.float32),
        pltpu.VMEM((DIM, CHUNK), jnp.float32),
        pltpu.SemaphoreType.DMA,
        pltpu.SemaphoreType.DMA,
        pltpu.SemaphoreType.DMA,
        pltpu.SemaphoreType.DMA,
    ],
    compiler_params=pltpu.CompilerParams(use_tc_tiling_on_sc=False,
                                         needs_layout_passes=False),
)
def _gather_kernel(idx_hbm, table_hbm, out_hbm,
                   idx0, idx1, rows0, rows1, t0, t1, sg0, sg1, so0, so1):
    wid = lax.axis_index("s") * NC + lax.axis_index("c")
    cbase = wid * C_PER_W

    def fetch(c, idx_v, rows_v, sg):
        off = pl.multiple_of(c * CHUNK, 8)
        pltpu.sync_copy(idx_hbm.at[pl.ds(off, CHUNK)], idx_v)
        pltpu.make_async_copy(table_hbm.at[idx_v], rows_v, sg).start()

    def wait_gather(idx_v, rows_v, sg):
        pltpu.make_async_copy(table_hbm.at[idx_v], rows_v, sg).wait()

    def transpose(rows_v, t_v):
        def fbody(f, carry):
            fvec = jnp.full((16,), 0, jnp.int32) + f
            for jb in range(CHUNK // 16):
                jvec = jb * 16 + lax.iota(jnp.int32, 16)
                t_v[f, pl.ds(jb * 16, 16)] = plsc.load_gather(rows_v, [jvec, fvec])
            return carry

        lax.fori_loop(0, DIM, fbody, 0)

    def wb_slice(c):
        s = c // CPR
        b0 = pl.multiple_of((c - s * CPR) * CHUNK, 8)
        return out_hbm.at[s, :, pl.ds(b0, CHUNK)]

    def wb_start(c, t_v, so):
        pltpu.make_async_copy(t_v, wb_slice(c), so).start()

    def wb_wait(c, t_v, so):
        pltpu.make_async_copy(t_v, wb_slice(c), so).wait()

    fetch(cbase, idx0, rows0, sg0)
    fetch(cbase + 1, idx1, rows1, sg1)
    wait_gather(idx0, rows0, sg0)
    transpose(rows0, t0)
    wb_start(cbase, t0, so0)
    wait_gather(idx1, rows1, sg1)
    transpose(rows1, t1)
    wb_start(cbase + 1, t1, so1)

    def body(i, carry):
        c0 = cbase + i * 2
        c1 = c0 + 1
        fetch(c0, idx0, rows0, sg0)
        fetch(c1, idx1, rows1, sg1)
        wait_gather(idx0, rows0, sg0)
        wb_wait(c0 - 2, t0, so0)
        transpose(rows0, t0)
        wb_start(c0, t0, so0)
        wait_gather(idx1, rows1, sg1)
        wb_wait(c1 - 2, t1, so1)
        transpose(rows1, t1)
        wb_start(c1, t1, so1)
        return carry

    lax.fori_loop(1, NPAIR, body, 0)
    wb_wait(cbase + C_PER_W - 2, t0, so0)
    wb_wait(cbase + C_PER_W - 1, t1, so1)


def kernel(input, weight):
    idx = input.reshape(-1).astype(jnp.int32)
    out = _gather_kernel(idx, weight)
    return out.transpose(0, 2, 1)


# speedup vs baseline: 1.7729x; 1.7729x over previous
import functools

import jax
import jax.numpy as jnp
from jax import lax
from jax.experimental import pallas as pl
from jax.experimental.pallas import tpu as pltpu
from jax.experimental.pallas import tpu_sc as plsc

SEQ_LEN, BATCH, DIM = 200, 4096, 64
TOTAL = SEQ_LEN * BATCH
NC, NS = 2, 16
NW = NC * NS
B_PER_W = TOTAL // NW
CHUNK = 800
NCHUNK = B_PER_W // CHUNK
NPAIR = NCHUNK // 2

_mesh = plsc.VectorSubcoreMesh(core_axis_name="c", subcore_axis_name="s")


@functools.partial(
    pl.kernel,
    out_type=jax.ShapeDtypeStruct((TOTAL, DIM), jnp.float32),
    mesh=_mesh,
    scratch_types=[
        pltpu.VMEM((CHUNK,), jnp.int32),
        pltpu.VMEM((CHUNK,), jnp.int32),
        pltpu.VMEM((CHUNK, DIM), jnp.float32),
        pltpu.VMEM((CHUNK, DIM), jnp.float32),
        pltpu.SemaphoreType.DMA,
        pltpu.SemaphoreType.DMA,
        pltpu.SemaphoreType.DMA,
        pltpu.SemaphoreType.DMA,
    ],
    compiler_params=pltpu.CompilerParams(use_tc_tiling_on_sc=False,
                                         needs_layout_passes=False),
)
def _gather_kernel(idx_hbm, table_hbm, out_hbm,
                   idx0, idx1, rows0, rows1, sg0, sg1, so0, so1):
    wid = lax.axis_index("s") * NC + lax.axis_index("c")
    base = wid * B_PER_W

    def off(c):
        return pl.multiple_of(base + c * CHUNK, 8)

    def fetch_and_gather(c, idx_v, rows_v, sg):
        pltpu.sync_copy(idx_hbm.at[pl.ds(off(c), CHUNK)], idx_v)
        pltpu.make_async_copy(table_hbm.at[idx_v], rows_v, sg).start()

    def finish_and_writeback(c, idx_v, rows_v, sg, so):
        pltpu.make_async_copy(table_hbm.at[idx_v], rows_v, sg).wait()
        pltpu.make_async_copy(rows_v, out_hbm.at[pl.ds(off(c), CHUNK)], so).start()

    def wait_writeback(c, rows_v, so):
        pltpu.make_async_copy(rows_v, out_hbm.at[pl.ds(off(c), CHUNK)], so).wait()

    fetch_and_gather(0, idx0, rows0, sg0)
    fetch_and_gather(1, idx1, rows1, sg1)
    finish_and_writeback(0, idx0, rows0, sg0, so0)
    finish_and_writeback(1, idx1, rows1, sg1, so1)

    def body(i, carry):
        c0 = i * 2
        c1 = c0 + 1
        wait_writeback(c0 - 2, rows0, so0)
        fetch_and_gather(c0, idx0, rows0, sg0)
        wait_writeback(c1 - 2, rows1, so1)
        fetch_and_gather(c1, idx1, rows1, sg1)
        finish_and_writeback(c0, idx0, rows0, sg0, so0)
        finish_and_writeback(c1, idx1, rows1, sg1, so1)
        return carry

    lax.fori_loop(1, NPAIR, body, 0)
    wait_writeback(NCHUNK - 2, rows0, so0)
    wait_writeback(NCHUNK - 1, rows1, so1)


def kernel(input, weight):
    idx = input.reshape(-1).astype(jnp.int32)
    out = _gather_kernel(idx, weight)
    return out.reshape(SEQ_LEN, BATCH, DIM)
